# Initial kernel scaffold; baseline (speedup 1.0000x reference)
import jax
import jax.numpy as jnp
from jax.experimental import pallas as pl


def kernel(x, W_ih, W_hh, b_ih, b_hh, W_fc, b_fc):
    raise NotImplementedError("write your pallas kernel here")


# speedup vs baseline: 2.5054x; 2.5054x over previous
import functools

import jax
import jax.numpy as jnp
from jax.experimental import pallas as pl
from jax.experimental.pallas import tpu as pltpu


def _lstm_kernel(x_ref, wc_ref, wfc_ref, out_ref, inp_ref, c_ref, *, T, H):
    inp_ref[8:8 + H, :] = jnp.zeros((H,) + inp_ref.shape[1:], jnp.float32)
    c_ref[...] = jnp.zeros(c_ref.shape, jnp.float32)

    def step(t, _):
        inp_ref[0:8, :] = x_ref[t]
        g = jnp.dot(wc_ref[...], inp_ref[...],
                    preferred_element_type=jnp.float32)
        i_g = jax.nn.sigmoid(g[0:H])
        f_g = jax.nn.sigmoid(g[H:2 * H])
        g_g = jnp.tanh(g[2 * H:3 * H])
        o_g = jax.nn.sigmoid(g[3 * H:4 * H])
        c = f_g * c_ref[...] + i_g * g_g
        c_ref[...] = c
        inp_ref[8:8 + H, :] = o_g * jnp.tanh(c)
        return ()

    jax.lax.fori_loop(0, T, step, (), unroll=False)

    out_ref[...] = jnp.dot(wfc_ref[...], inp_ref[...],
                           preferred_element_type=jnp.float32)


def kernel(x, W_ih, W_hh, b_ih, b_hh, W_fc, b_fc):
    B, T, I = x.shape
    H = W_hh.shape[1]
    O = W_fc.shape[0]
    K = 8 + H

    xT = jnp.transpose(x, (1, 2, 0))
    pad = jnp.concatenate(
        [jnp.ones((T, 1, B), x.dtype), jnp.zeros((T, 8 - I - 1, B), x.dtype)],
        axis=1)
    xTp = jnp.concatenate([xT, pad], axis=1)

    Wc = jnp.zeros((4 * H, K), jnp.float32)
    Wc = Wc.at[:, 0:I].set(W_ih)
    Wc = Wc.at[:, I].set(b_ih + b_hh)
    Wc = Wc.at[:, 8:K].set(W_hh)

    Wfcp = jnp.zeros((8, K), jnp.float32)
    Wfcp = Wfcp.at[0:O, 8:K].set(W_fc)
    Wfcp = Wfcp.at[0:O, I].set(b_fc)

    block_B = min(1024, B)
    grid = (B // block_B,)

    out = pl.pallas_call(
        functools.partial(_lstm_kernel, T=T, H=H),
        out_shape=jax.ShapeDtypeStruct((8, B), jnp.float32),
        grid=grid,
        in_specs=[
            pl.BlockSpec((T, 8, block_B), lambda i: (0, 0, i)),
            pl.BlockSpec((4 * H, K), lambda i: (0, 0)),
            pl.BlockSpec((8, K), lambda i: (0, 0)),
        ],
        out_specs=pl.BlockSpec((8, block_B), lambda i: (0, i)),
        scratch_shapes=[
            pltpu.VMEM((K, block_B), jnp.float32),
            pltpu.VMEM((H, block_B), jnp.float32),
        ],
        compiler_params=pltpu.CompilerParams(
            dimension_semantics=("parallel",),
            vmem_limit_bytes=48 * 1024 * 1024,
        ),
        name="fatigue_lstm",
    )(xTp, Wc, Wfcp)

    return out[:O].T


# speedup vs baseline: 2.9738x; 1.1870x over previous
import functools

import jax
import jax.numpy as jnp
from jax.experimental import pallas as pl
from jax.experimental.pallas import tpu as pltpu


def _sig(x):
    return 0.5 * jnp.tanh(0.5 * x) + 0.5


def _lstm_kernel(x_ref, wc_ref, wfc_ref, out_ref, inp_a, c_a, inp_b, c_b,
                 *, T, H, HB):
    for inp_ref, c_ref in ((inp_a, c_a), (inp_b, c_b)):
        inp_ref[8:8 + H, :] = jnp.zeros((H, HB), jnp.float32)
        c_ref[...] = jnp.zeros((H, HB), jnp.float32)

    def half(t, inp_ref, c_ref, lo):
        inp_ref[0:8, :] = x_ref[t, :, lo:lo + HB]
        g = jnp.dot(wc_ref[...], inp_ref[...],
                    preferred_element_type=jnp.float32)
        i_g = _sig(g[0:H])
        f_g = _sig(g[H:2 * H])
        g_g = jnp.tanh(g[2 * H:3 * H])
        o_g = _sig(g[3 * H:4 * H])
        c = f_g * c_ref[...] + i_g * g_g
        c_ref[...] = c
        inp_ref[8:8 + H, :] = o_g * jnp.tanh(c)

    def step(t, _):
        half(t, inp_a, c_a, 0)
        half(t, inp_b, c_b, HB)
        return ()

    jax.lax.fori_loop(0, T, step, (), unroll=False)

    out_ref[:, 0:HB] = jnp.dot(wfc_ref[...], inp_a[...],
                               preferred_element_type=jnp.float32)
    out_ref[:, HB:2 * HB] = jnp.dot(wfc_ref[...], inp_b[...],
                                    preferred_element_type=jnp.float32)


def kernel(x, W_ih, W_hh, b_ih, b_hh, W_fc, b_fc):
    B, T, I = x.shape
    H = W_hh.shape[1]
    O = W_fc.shape[0]
    K = 8 + H

    xT = jnp.transpose(x, (1, 2, 0))
    pad = jnp.concatenate(
        [jnp.ones((T, 1, B), x.dtype), jnp.zeros((T, 8 - I - 1, B), x.dtype)],
        axis=1)
    xTp = jnp.concatenate([xT, pad], axis=1)

    Wc = jnp.zeros((4 * H, K), jnp.float32)
    Wc = Wc.at[:, 0:I].set(W_ih)
    Wc = Wc.at[:, I].set(b_ih + b_hh)
    Wc = Wc.at[:, 8:K].set(W_hh)

    Wfcp = jnp.zeros((8, K), jnp.float32)
    Wfcp = Wfcp.at[0:O, 8:K].set(W_fc)
    Wfcp = Wfcp.at[0:O, I].set(b_fc)

    block_B = min(1024, B)
    HB = block_B // 2
    grid = (B // block_B,)

    out = pl.pallas_call(
        functools.partial(_lstm_kernel, T=T, H=H, HB=HB),
        out_shape=jax.ShapeDtypeStruct((8, B), jnp.float32),
        grid=grid,
        in_specs=[
            pl.BlockSpec((T, 8, block_B), lambda i: (0, 0, i)),
            pl.BlockSpec((4 * H, K), lambda i: (0, 0)),
            pl.BlockSpec((8, K), lambda i: (0, 0)),
        ],
        out_specs=pl.BlockSpec((8, block_B), lambda i: (0, i)),
        scratch_shapes=[
            pltpu.VMEM((K, HB), jnp.float32),
            pltpu.VMEM((H, HB), jnp.float32),
            pltpu.VMEM((K, HB), jnp.float32),
            pltpu.VMEM((H, HB), jnp.float32),
        ],
        compiler_params=pltpu.CompilerParams(
            dimension_semantics=("parallel",),
            vmem_limit_bytes=48 * 1024 * 1024,
        ),
        name="fatigue_lstm",
    )(xTp, Wc, Wfcp)

    return out[:O].T


# speedup vs baseline: 3.2721x; 1.1003x over previous
import functools

import jax
import jax.numpy as jnp
from jax.experimental import pallas as pl
from jax.experimental.pallas import tpu as pltpu

_NC = 8


def _lstm_kernel(xs_ref, x0_ref, wc_ref, wfc_ref, out_ref, *scratch,
                 Tc, NT, H, HB):
    inps = scratch[0:_NC]
    cs = scratch[_NC:2 * _NC]
    gs = scratch[2 * _NC:3 * _NC]
    j = pl.program_id(1)

    @pl.when(j == 0)
    def _prologue():
        for k in range(_NC):
            inps[k][8:8 + H, :] = jnp.zeros((H, HB), jnp.float32)
            cs[k][...] = jnp.zeros((H, HB), jnp.float32)
            inps[k][0:8, :] = x0_ref[0, :, k * HB:(k + 1) * HB]
            gs[k][...] = jnp.dot(wc_ref[...], inps[k][...],
                                 preferred_element_type=jnp.float32)

    def step(t, _):
        for k in range(_NC):
            inp_ref, c_ref, g_ref = inps[k], cs[k], gs[k]
            g = g_ref[...]
            ti = jnp.tanh(g[0:H])
            tf = jnp.tanh(g[H:2 * H])
            gg = jnp.tanh(g[2 * H:3 * H])
            to = jnp.tanh(g[3 * H:4 * H])
            c_old = c_ref[...]
            c = 0.5 * ((tf * c_old + c_old) + (ti * gg + gg))
            c_ref[...] = c
            tc = jnp.tanh(c)
            inp_ref[8:8 + H, :] = 0.5 * (to * tc + tc)
            inp_ref[0:8, :] = xs_ref[t, :, k * HB:(k + 1) * HB]
            g_ref[...] = jnp.dot(wc_ref[...], inp_ref[...],
                                 preferred_element_type=jnp.float32)
        return ()

    jax.lax.fori_loop(0, Tc, step, (), unroll=False)

    @pl.when(j == NT - 1)
    def _epilogue():
        for k in range(_NC):
            out_ref[:, k * HB:(k + 1) * HB] = jnp.dot(
                wfc_ref[...], inps[k][...],
                preferred_element_type=jnp.float32)


def kernel(x, W_ih, W_hh, b_ih, b_hh, W_fc, b_fc):
    B, T, I = x.shape
    H = W_hh.shape[1]
    O = W_fc.shape[0]
    K = 8 + H

    xT = jnp.transpose(x, (1, 2, 0))
    xT = jnp.concatenate([xT, jnp.zeros((1, I, B), x.dtype)], axis=0)
    pad = jnp.concatenate(
        [jnp.ones((T + 1, 1, B), x.dtype),
         jnp.zeros((T + 1, 8 - I - 1, B), x.dtype)], axis=1)
    xTp = jnp.concatenate([xT, pad], axis=1)
    x_shift = xTp[1:]
    x0 = xTp[0:1]

    Wc = jnp.zeros((4 * H, K), jnp.float32)
    Wc = Wc.at[:, 0:I].set(W_ih)
    Wc = Wc.at[:, I].set(b_ih + b_hh)
    Wc = Wc.at[:, 8:K].set(W_hh)
    gate_scale = jnp.concatenate(
        [jnp.full((2 * H, 1), 0.5, jnp.float32),
         jnp.ones((H, 1), jnp.float32),
         jnp.full((H, 1), 0.5, jnp.float32)], axis=0)
    Wc = Wc * gate_scale

    Wfcp = jnp.zeros((8, K), jnp.float32)
    Wfcp = Wfcp.at[0:O, 8:K].set(W_fc)
    Wfcp = Wfcp.at[0:O, I].set(b_fc)

    block_B = min(2048, B)
    HB = block_B // _NC
    Tc = 64
    NT = T // Tc
    grid = (B // block_B, NT)

    out = pl.pallas_call(
        functools.partial(_lstm_kernel, Tc=Tc, NT=NT, H=H, HB=HB),
        out_shape=jax.ShapeDtypeStruct((8, B), jnp.float32),
        grid=grid,
        in_specs=[
            pl.BlockSpec((Tc, 8, block_B), lambda i, j: (j, 0, i)),
            pl.BlockSpec((1, 8, block_B), lambda i, j: (0, 0, i)),
            pl.BlockSpec((4 * H, K), lambda i, j: (0, 0)),
            pl.BlockSpec((8, K), lambda i, j: (0, 0)),
        ],
        out_specs=pl.BlockSpec((8, block_B), lambda i, j: (0, i)),
        scratch_shapes=(
            [pltpu.VMEM((K, HB), jnp.float32) for _ in range(_NC)]
            + [pltpu.VMEM((H, HB), jnp.float32) for _ in range(_NC)]
            + [pltpu.VMEM((4 * H, HB), jnp.float32) for _ in range(_NC)]
        ),
        compiler_params=pltpu.CompilerParams(
            dimension_semantics=("parallel", "arbitrary"),
            vmem_limit_bytes=48 * 1024 * 1024,
        ),
        name="fatigue_lstm",
    )(x_shift, x0, Wc, Wfcp)

    return out[:O].T


# speedup vs baseline: 3.9407x; 1.2043x over previous
import functools

import jax
import jax.numpy as jnp
from jax.experimental import pallas as pl
from jax.experimental.pallas import tpu as pltpu

_NC = 8


def _ew(g_ref, c_ref, inp_ref, H):
    g = g_ref[...]
    ti = jnp.tanh(g[0:H])
    tf = jnp.tanh(g[H:2 * H])
    gg = jnp.tanh(g[2 * H:3 * H])
    to = jnp.tanh(g[3 * H:4 * H])
    c_old = c_ref[...]
    c = 0.5 * ((tf * c_old + c_old) + (ti * gg + gg))
    c_ref[...] = c
    tc = jnp.tanh(c)
    inp_ref[8:8 + H, :] = 0.5 * (to * tc + tc)


def _lstm_kernel(xs_ref, wc_ref, wfc_ref, out_ref, *scratch,
                 Tc, NT, H, HB, I):
    inps = scratch[0:_NC]
    cs = scratch[_NC:2 * _NC]
    gs = scratch[2 * _NC:3 * _NC]
    j = pl.program_id(1)

    ones_row = jnp.where(
        jax.lax.broadcasted_iota(jnp.int32, (8 - I, HB), 0) == 0,
        jnp.float32(1.0), jnp.float32(0.0))

    @pl.when(j == 0)
    def _prologue():
        for k in range(_NC):
            inps[k][I:8, :] = ones_row
            cs[k][...] = jnp.zeros((H, HB), jnp.float32)
            gs[k][...] = jnp.zeros((4 * H, HB), jnp.float32)

    def step(t, _):
        for k in range(_NC):
            inp_ref, c_ref, g_ref = inps[k], cs[k], gs[k]
            _ew(g_ref, c_ref, inp_ref, H)
            inp_ref[0:I, :] = xs_ref[t, :, k * HB:(k + 1) * HB]
            g_ref[...] = jnp.dot(wc_ref[...], inp_ref[...],
                                 preferred_element_type=jnp.float32)
        return ()

    jax.lax.fori_loop(0, Tc, step, (), unroll=False)

    @pl.when(j == NT - 1)
    def _epilogue():
        for k in range(_NC):
            _ew(gs[k], cs[k], inps[k], H)
            out_ref[:, k * HB:(k + 1) * HB] = jnp.dot(
                wfc_ref[...], inps[k][...],
                preferred_element_type=jnp.float32)


def kernel(x, W_ih, W_hh, b_ih, b_hh, W_fc, b_fc):
    B, T, I = x.shape
    H = W_hh.shape[1]
    O = W_fc.shape[0]
    K = 8 + H

    xT = jnp.transpose(x, (1, 2, 0))

    Wc = jnp.zeros((4 * H, K), jnp.float32)
    Wc = Wc.at[:, 0:I].set(W_ih)
    Wc = Wc.at[:, I].set(b_ih + b_hh)
    Wc = Wc.at[:, 8:K].set(W_hh)
    gate_scale = jnp.concatenate(
        [jnp.full((2 * H, 1), 0.5, jnp.float32),
         jnp.ones((H, 1), jnp.float32),
         jnp.full((H, 1), 0.5, jnp.float32)], axis=0)
    Wc = Wc * gate_scale

    Wfcp = jnp.zeros((8, K), jnp.float32)
    Wfcp = Wfcp.at[0:O, 8:K].set(W_fc)
    Wfcp = Wfcp.at[0:O, I].set(b_fc)

    block_B = min(2048, B)
    HB = block_B // _NC
    Tc = 64
    NT = T // Tc
    grid = (B // block_B, NT)

    out = pl.pallas_call(
        functools.partial(_lstm_kernel, Tc=Tc, NT=NT, H=H, HB=HB, I=I),
        out_shape=jax.ShapeDtypeStruct((8, B), jnp.float32),
        grid=grid,
        in_specs=[
            pl.BlockSpec((Tc, I, block_B), lambda i, j: (j, 0, i)),
            pl.BlockSpec((4 * H, K), lambda i, j: (0, 0)),
            pl.BlockSpec((8, K), lambda i, j: (0, 0)),
        ],
        out_specs=pl.BlockSpec((8, block_B), lambda i, j: (0, i)),
        scratch_shapes=(
            [pltpu.VMEM((K, HB), jnp.float32) for _ in range(_NC)]
            + [pltpu.VMEM((H, HB), jnp.float32) for _ in range(_NC)]
            + [pltpu.VMEM((4 * H, HB), jnp.float32) for _ in range(_NC)]
        ),
        compiler_params=pltpu.CompilerParams(
            dimension_semantics=("parallel", "arbitrary"),
            vmem_limit_bytes=48 * 1024 * 1024,
        ),
        name="fatigue_lstm",
    )(xT, Wc, Wfcp)

    return out[:O].T


# speedup vs baseline: 5.6830x; 1.4422x over previous
import functools

import jax
import jax.numpy as jnp
from jax.experimental import pallas as pl
from jax.experimental.pallas import tpu as pltpu

_NC = 8


def _ew(g_ref, c_ref, inp_ref, H):
    g = g_ref[...]
    ti = jnp.tanh(g[0:H])
    tf = jnp.tanh(g[H:2 * H])
    gg = jnp.tanh(g[2 * H:3 * H])
    to = jnp.tanh(g[3 * H:4 * H])
    c_old = c_ref[...]
    c = 0.5 * ((tf * c_old + c_old) + (ti * gg + gg))
    c_ref[...] = c
    tc = jnp.tanh(c)
    inp_ref[8:8 + H, :] = (tc * (to + 1.0)).astype(jnp.bfloat16)


def _lstm_kernel(xs_ref, wc_ref, wfc_ref, out_ref, *scratch,
                 Tc, NT, H, HB):
    inps = scratch[0:_NC]
    cs = scratch[_NC:2 * _NC]
    gs = scratch[2 * _NC:3 * _NC]
    j = pl.program_id(1)

    @pl.when(j == 0)
    def _prologue():
        for k in range(_NC):
            cs[k][...] = jnp.zeros((H, HB), jnp.float32)
            gs[k][...] = jnp.zeros((4 * H, HB), jnp.float32)

    def step(t, _):
        for k in range(_NC):
            inp_ref, c_ref, g_ref = inps[k], cs[k], gs[k]
            _ew(g_ref, c_ref, inp_ref, H)
            inp_ref[0:8, :] = xs_ref[t, :, k * HB:(k + 1) * HB]
            g_ref[...] = jnp.dot(wc_ref[...], inp_ref[...],
                                 preferred_element_type=jnp.float32)
        return ()

    jax.lax.fori_loop(0, Tc, step, (), unroll=16)

    @pl.when(j == NT - 1)
    def _epilogue():
        for k in range(_NC):
            _ew(gs[k], cs[k], inps[k], H)
            out_ref[:, k * HB:(k + 1) * HB] = jnp.dot(
                wfc_ref[...], inps[k][...],
                preferred_element_type=jnp.float32)


def kernel(x, W_ih, W_hh, b_ih, b_hh, W_fc, b_fc):
    B, T, I = x.shape
    H = W_hh.shape[1]
    O = W_fc.shape[0]
    K = 8 + H

    xT = jnp.transpose(x.astype(jnp.bfloat16), (1, 2, 0))
    pad = jnp.concatenate(
        [jnp.ones((T, 1, B), jnp.bfloat16),
         jnp.zeros((T, 8 - I - 1, B), jnp.bfloat16)], axis=1)
    xTp = jnp.concatenate([xT, pad], axis=1)

    Wc = jnp.zeros((4 * H, K), jnp.float32)
    Wc = Wc.at[:, 0:I].set(W_ih)
    Wc = Wc.at[:, I].set(b_ih + b_hh)
    Wc = Wc.at[:, 8:K].set(W_hh * 0.5)
    gate_scale = jnp.concatenate(
        [jnp.full((2 * H, 1), 0.5, jnp.float32),
         jnp.ones((H, 1), jnp.float32),
         jnp.full((H, 1), 0.5, jnp.float32)], axis=0)
    Wc = (Wc * gate_scale).astype(jnp.bfloat16)

    Wfcp = jnp.zeros((8, K), jnp.float32)
    Wfcp = Wfcp.at[0:O, 8:K].set(W_fc * 0.5)
    Wfcp = Wfcp.at[0:O, I].set(b_fc)
    Wfcp = Wfcp.astype(jnp.bfloat16)

    block_B = min(2048, B)
    HB = block_B // _NC
    Tc = 64
    NT = T // Tc
    grid = (B // block_B, NT)

    out = pl.pallas_call(
        functools.partial(_lstm_kernel, Tc=Tc, NT=NT, H=H, HB=HB),
        out_shape=jax.ShapeDtypeStruct((8, B), jnp.float32),
        grid=grid,
        in_specs=[
            pl.BlockSpec((Tc, 8, block_B), lambda i, j: (j, 0, i)),
            pl.BlockSpec((4 * H, K), lambda i, j: (0, 0)),
            pl.BlockSpec((8, K), lambda i, j: (0, 0)),
        ],
        out_specs=pl.BlockSpec((8, block_B), lambda i, j: (0, i)),
        scratch_shapes=(
            [pltpu.VMEM((K, HB), jnp.bfloat16) for _ in range(_NC)]
            + [pltpu.VMEM((H, HB), jnp.float32) for _ in range(_NC)]
            + [pltpu.VMEM((4 * H, HB), jnp.float32) for _ in range(_NC)]
        ),
        compiler_params=pltpu.CompilerParams(
            dimension_semantics=("parallel", "arbitrary"),
            vmem_limit_bytes=48 * 1024 * 1024,
        ),
        name="fatigue_lstm",
    )(xTp, Wc, Wfcp)

    return out[:O].T


# speedup vs baseline: 6.0448x; 1.0637x over previous
import functools

import jax
import jax.numpy as jnp
from jax.experimental import pallas as pl
from jax.experimental.pallas import tpu as pltpu

_NC = 8


def _lstm_kernel(xs_ref, wc_ref, wfc_ref, out_ref, *scratch,
                 Tc, NT, H, HB, I):
    inps = scratch[0:_NC]
    cs = scratch[_NC:2 * _NC]
    j = pl.program_id(1)

    ones_row = jnp.where(
        jax.lax.broadcasted_iota(jnp.int32, (8 - I, HB), 0) == 0,
        jnp.float32(1.0), jnp.float32(0.0))

    @pl.when(j == 0)
    def _prologue():
        for k in range(_NC):
            inps[k][I:8, :] = ones_row
            inps[k][8:8 + H, :] = jnp.zeros((H, HB), jnp.float32)
            cs[k][...] = jnp.zeros((H, HB), jnp.float32)

    def step(t, _):
        for k in range(_NC):
            inp_ref, c_ref = inps[k], cs[k]
            inp_ref[0:I, :] = xs_ref[t, :, k * HB:(k + 1) * HB]
            g = jnp.dot(wc_ref[...], inp_ref[...],
                        preferred_element_type=jnp.float32)
            ti = jnp.tanh(g[0:H])
            tf = jnp.tanh(g[H:2 * H])
            gg = jnp.tanh(g[2 * H:3 * H])
            to = jnp.tanh(g[3 * H:4 * H])
            c_old = c_ref[...]
            c = 0.5 * ((tf * c_old + c_old) + (ti * gg + gg))
            c_ref[...] = c
            tc = jnp.tanh(c)
            inp_ref[8:8 + H, :] = 0.5 * (to * tc + tc)
        return ()

    jax.lax.fori_loop(0, Tc, step, (), unroll=8)

    @pl.when(j == NT - 1)
    def _epilogue():
        for k in range(_NC):
            out_ref[:, k * HB:(k + 1) * HB] = jnp.dot(
                wfc_ref[...], inps[k][...],
                preferred_element_type=jnp.float32)


def kernel(x, W_ih, W_hh, b_ih, b_hh, W_fc, b_fc):
    B, T, I = x.shape
    H = W_hh.shape[1]
    O = W_fc.shape[0]
    K = 8 + H

    xT = jnp.transpose(x, (1, 2, 0))

    Wc = jnp.zeros((4 * H, K), jnp.float32)
    Wc = Wc.at[:, 0:I].set(W_ih)
    Wc = Wc.at[:, I].set(b_ih + b_hh)
    Wc = Wc.at[:, 8:K].set(W_hh)
    gate_scale = jnp.concatenate(
        [jnp.full((2 * H, 1), 0.5, jnp.float32),
         jnp.ones((H, 1), jnp.float32),
         jnp.full((H, 1), 0.5, jnp.float32)], axis=0)
    Wc = Wc * gate_scale

    Wfcp = jnp.zeros((8, K), jnp.float32)
    Wfcp = Wfcp.at[0:O, 8:K].set(W_fc)
    Wfcp = Wfcp.at[0:O, I].set(b_fc)

    block_B = min(2048, B)
    HB = block_B // _NC
    Tc = 64
    NT = T // Tc
    grid = (B // block_B, NT)

    out = pl.pallas_call(
        functools.partial(_lstm_kernel, Tc=Tc, NT=NT, H=H, HB=HB, I=I),
        out_shape=jax.ShapeDtypeStruct((8, B), jnp.float32),
        grid=grid,
        in_specs=[
            pl.BlockSpec((Tc, I, block_B), lambda i, j: (j, 0, i)),
            pl.BlockSpec((4 * H, K), lambda i, j: (0, 0)),
            pl.BlockSpec((8, K), lambda i, j: (0, 0)),
        ],
        out_specs=pl.BlockSpec((8, block_B), lambda i, j: (0, i)),
        scratch_shapes=(
            [pltpu.VMEM((K, HB), jnp.float32) for _ in range(_NC)]
            + [pltpu.VMEM((H, HB), jnp.float32) for _ in range(_NC)]
        ),
        compiler_params=pltpu.CompilerParams(
            dimension_semantics=("parallel", "arbitrary"),
            vmem_limit_bytes=48 * 1024 * 1024,
        ),
        name="fatigue_lstm",
    )(xT, Wc, Wfcp)

    return out[:O].T
